# speedup vs baseline: 1.3899x; 1.3899x over previous
import functools

import jax
import jax.numpy as jnp
from jax.experimental import pallas as pl
from jax.experimental.pallas import tpu as pltpu

N = 100000
D = 128
E = 100000
MSG = 16
TD = 16

BR = 2000


def _dense_body(mem_ref, s_ref, wcat_ref, wnbr_ref, b_ref, h_ref, a_ref, b2_ref):
    mem = mem_ref[...]
    c3 = jnp.dot(mem, wcat_ref[...], preferred_element_type=jnp.float32)
    hpre = c3[:, :D] + jnp.dot(s_ref[...], wnbr_ref[...],
                               preferred_element_type=jnp.float32) + b_ref[...]
    h_ref[...] = jnp.tanh(hpre)
    a_ref[...] = c3[:, D:2 * D]
    b2_ref[...] = c3[:, 2 * D:3 * D]


def _dense_phase(memory, S, Wcat, W_nbr, b):
    grid = (N // BR,)
    return pl.pallas_call(
        _dense_body,
        grid=grid,
        in_specs=[
            pl.BlockSpec((BR, D), lambda i: (i, 0)),
            pl.BlockSpec((BR, D), lambda i: (i, 0)),
            pl.BlockSpec((D, 3 * D), lambda i: (0, 0)),
            pl.BlockSpec((D, D), lambda i: (0, 0)),
            pl.BlockSpec((1, D), lambda i: (0, 0)),
        ],
        out_specs=[
            pl.BlockSpec((BR, D), lambda i: (i, 0)),
            pl.BlockSpec((BR, D), lambda i: (i, 0)),
            pl.BlockSpec((BR, D), lambda i: (i, 0)),
        ],
        out_shape=[jax.ShapeDtypeStruct((N, D), jnp.float32)] * 3,
    )(memory, S, Wcat, W_nbr, b)


def _final_body(a_ref, b2g_ref, msgg_ref, dtw_ref, valid_ref, mem_ref,
                wu3_ref, wu4_ref, bupd_ref, wt_ref, out_ref):
    te = jnp.cos(dtw_ref[...] * wt_ref[...])
    pre = (a_ref[...] + b2g_ref[...]
           + jnp.dot(msgg_ref[...], wu3_ref[...], preferred_element_type=jnp.float32)
           + jnp.dot(te, wu4_ref[...], preferred_element_type=jnp.float32)
           + bupd_ref[...])
    m = jnp.tanh(pre)
    out_ref[...] = jnp.where(valid_ref[...] > 0, m, mem_ref[...])


def _final_phase(A, B2g, msg_g, dtw, valid, memory, Wu3, Wu4, b_upd, w_time):
    grid = (N // BR,)
    return pl.pallas_call(
        _final_body,
        grid=grid,
        in_specs=[
            pl.BlockSpec((BR, D), lambda i: (i, 0)),
            pl.BlockSpec((BR, D), lambda i: (i, 0)),
            pl.BlockSpec((BR, MSG), lambda i: (i, 0)),
            pl.BlockSpec((BR, 1), lambda i: (i, 0)),
            pl.BlockSpec((BR, 1), lambda i: (i, 0)),
            pl.BlockSpec((BR, D), lambda i: (i, 0)),
            pl.BlockSpec((MSG, D), lambda i: (0, 0)),
            pl.BlockSpec((TD, D), lambda i: (0, 0)),
            pl.BlockSpec((1, D), lambda i: (0, 0)),
            pl.BlockSpec((1, TD), lambda i: (0, 0)),
        ],
        out_specs=pl.BlockSpec((BR, D), lambda i: (i, 0)),
        out_shape=jax.ShapeDtypeStruct((N, D), jnp.float32),
    )(A, B2g, msg_g, dtw, valid, memory, Wu3, Wu4, b_upd, w_time)


def kernel(edge_index, t, msg, memory, last_update, W_self, W_nbr, b, W_upd, b_upd, w_time):
    src, dst = edge_index[0], edge_index[1]
    Wcat = jnp.concatenate([W_self, W_upd[:D], W_upd[D:2 * D]], axis=1)
    Wu3 = W_upd[2 * D:2 * D + MSG]
    Wu4 = W_upd[2 * D + MSG:]

    S = jnp.zeros((N, D), jnp.float32).at[dst].add(memory[src])

    H, A, B2 = _dense_phase(memory, S, Wcat, W_nbr, b[None, :])

    h_src = H[src]
    h_dst = H[dst]

    p = jnp.full((N,), -1, jnp.int32)
    p = p.at[src].max(jnp.arange(E, dtype=jnp.int32))
    p = p.at[dst].max(jnp.arange(E, dtype=jnp.int32) + E)

    valid = (p >= 0).astype(jnp.int32)
    ew = jnp.where(p >= E, p - E, p)
    ew_c = jnp.clip(ew, 0, E - 1)
    opp = jnp.where(p >= E, src[ew_c], dst[ew_c])
    dtw = t[ew_c] - last_update
    new_lu = jnp.where(valid > 0, t[ew_c], last_update)
    msg_g = msg[ew_c]
    B2g = B2[opp]

    new_memory = _final_phase(A, B2g, msg_g, dtw[:, None], valid[:, None],
                              memory, Wu3, Wu4, b_upd[None, :], w_time[None, :])

    return (h_src, h_dst, new_memory, new_lu)
